# baseline (speedup 1.0000x reference)
import functools

import jax
import jax.numpy as jnp
from jax import lax
from jax.experimental import pallas as pl
from jax.experimental.pallas import tpu as pltpu
from jax.experimental.pallas import tpu_sc as plsc

HIDDEN = 1024
ROUTES = 8
BITS = 4
NGRAM = 4
ALPHA = 2 ** BITS
EMBED = 128
ROWS = ROUTES * ALPHA ** NGRAM

NUM_CORES = 2
NUM_SUBCORES = 16
NW = NUM_CORES * NUM_SUBCORES
LANES = 16

TILE_POS = 128
QPOS = 32


_RBLK = 1024
_HALO = 8


def _codes_of(logits, sel, gw):
    cb = 1.0 / (1.0 + jnp.exp(-jnp.abs(logits)))
    logcb = jnp.log(cb)
    bits = (logits > 0.0).astype(jnp.float32)
    logconf = jnp.dot(logcb, sel, preferred_element_type=jnp.float32)
    codes = jnp.dot(bits, gw, preferred_element_type=jnp.float32)
    return logconf, codes


def _route_body(x_ref, xh_ref, wr_ref, conf_ref, idx_ref, *, seq):
    T = x_ref.shape[0]
    h = pl.program_id(0) % (seq // _RBLK)
    row = lax.broadcasted_iota(jnp.int32, (ROUTES * BITS, ROUTES), 0)
    col = lax.broadcasted_iota(jnp.int32, (ROUTES * BITS, ROUTES), 1)
    sel = (row // BITS == col).astype(jnp.float32)
    gw = sel * (2.0 ** (row % BITS).astype(jnp.float32))
    logits = jnp.dot(x_ref[...], wr_ref[...],
                     preferred_element_type=jnp.float32)
    logconf, codes = _codes_of(logits, sel, gw)
    hlogits = jnp.dot(xh_ref[...], wr_ref[...],
                      preferred_element_type=jnp.float32)
    _, hcodes = _codes_of(hlogits, sel, gw)
    hcodes = hcodes * jnp.where(h == 0, 0.0, 1.0)
    conf_t = jnp.exp(jnp.transpose(logconf, (1, 0)))
    codes_t = jnp.transpose(codes, (1, 0))
    hcodes_t = jnp.transpose(hcodes, (1, 0))[:, _HALO - NGRAM + 1:]
    ext = jnp.concatenate([hcodes_t, codes_t], axis=1)
    addr = codes_t
    for k in range(1, NGRAM):
        addr = addr + ext[:, NGRAM - 1 - k:NGRAM - 1 - k + T] * float(ALPHA ** k)
    route_off = lax.broadcasted_iota(jnp.int32, (ROUTES, T), 0) * (ALPHA ** NGRAM)
    conf_ref[...] = conf_t
    idx_ref[...] = addr.astype(jnp.int32) + route_off


def _routing(x2, w_route, batch, seq):
    nblk = batch * seq // _RBLK
    per_batch = seq // _RBLK

    def _halo_map(i):
        return (jnp.maximum(i * _RBLK - _HALO, 0) // _HALO, 0)

    return pl.pallas_call(
        functools.partial(_route_body, seq=seq),
        grid=(nblk,),
        in_specs=[
            pl.BlockSpec((_RBLK, HIDDEN), lambda i: (i, 0)),
            pl.BlockSpec((_HALO, HIDDEN), _halo_map),
            pl.BlockSpec((HIDDEN, ROUTES * BITS), lambda i: (0, 0)),
        ],
        out_specs=[
            pl.BlockSpec((ROUTES, _RBLK), lambda i: (0, i)),
            pl.BlockSpec((ROUTES, _RBLK), lambda i: (0, i)),
        ],
        out_shape=[
            jax.ShapeDtypeStruct((ROUTES, batch * seq), jnp.float32),
            jax.ShapeDtypeStruct((ROUTES, batch * seq), jnp.int32),
        ],
    )(x2, x2, w_route)


_GDN = lax.GatherDimensionNumbers(
    offset_dims=(), collapsed_slice_dims=(0,), start_index_map=(0,))


def _lane_broadcast(v, lane):
    idx = jnp.full((LANES, 1), lane, jnp.int32)
    return lax.gather(v, idx, dimension_numbers=_GDN, slice_sizes=(1,),
                      mode=lax.GatherScatterMode.PROMISE_IN_BOUNDS)


def _make_pool_kernel(num_pos):
    pos_w = num_pos // NW
    nrow_w = pos_w * ROUTES // TILE_POS
    nq = pos_w // QPOS
    qper = TILE_POS // QPOS

    mesh = plsc.VectorSubcoreMesh(
        core_axis_name="c", subcore_axis_name="s",
        num_cores=NUM_CORES, num_subcores=NUM_SUBCORES)

    @functools.partial(
        pl.kernel, mesh=mesh,
        out_type=jax.ShapeDtypeStruct((num_pos, EMBED), jnp.float32),
        scratch_types=[
            pltpu.VMEM((nrow_w, TILE_POS), jnp.int32),
            pltpu.VMEM((nrow_w, TILE_POS), jnp.float32),
            [[pltpu.VMEM((QPOS, EMBED), jnp.float32)] * ROUTES] * 2,
            [pltpu.VMEM((QPOS, EMBED), jnp.float32)] * 2,
            [pltpu.SemaphoreType.DMA] * 2,
            [pltpu.SemaphoreType.DMA] * 2,
        ],
    )
    def pool_kernel(idx_hbm, conf_hbm, table_hbm, out_hbm,
                    idx_v, conf_v, rows, pools, gsems, ssems):
        wid = lax.axis_index("s") * NUM_CORES + lax.axis_index("c")
        pltpu.sync_copy(idx_hbm.at[pl.ds(wid * nrow_w, nrow_w)], idx_v)
        pltpu.sync_copy(conf_hbm.at[pl.ds(wid * nrow_w, nrow_w)], conf_v)

        def _gather_quarter(qt, par):
            ct = lax.div(qt, qper)
            off = lax.rem(qt, qper) * QPOS
            for r in range(ROUTES):
                pltpu.async_copy(
                    table_hbm.at[idx_v.at[ct * ROUTES + r, pl.ds(off, QPOS)]],
                    rows[par][r], gsems[par])

        def _drain_quarter(qt, par):
            ct = lax.div(qt, qper)
            off = lax.rem(qt, qper) * QPOS
            for r in range(ROUTES):
                pltpu.make_async_copy(
                    table_hbm.at[idx_v.at[ct * ROUTES + r, pl.ds(off, QPOS)]],
                    rows[par][r], gsems[par]).wait()

        def _out_slice(qt):
            return out_hbm.at[pl.ds(wid * pos_w + qt * QPOS, QPOS)]

        _gather_quarter(0, 0)

        @pl.loop(0, nq, step=2)
        def _qpair(g):
            for par in range(2):
                qt = g + par
                ct = lax.div(qt, qper)
                off = lax.rem(qt, qper) * QPOS

                @pl.when(qt + 1 < nq)
                def _():
                    _gather_quarter(qt + 1, 1 - par)

                _drain_quarter(qt, par)

                @pl.when(qt >= 2)
                def _():
                    pltpu.make_async_copy(pools[par], _out_slice(qt),
                                          ssems[par]).wait()

                @pl.loop(0, QPOS // LANES)
                def _pgroup(pp):
                    cvs = [conf_v[ct * ROUTES + r,
                                  pl.ds(off + pp * LANES, LANES)]
                           for r in range(ROUTES)]

                    @pl.loop(0, LANES)
                    def _pos(q):
                        p = pp * LANES + q
                        accs = [None] * (EMBED // LANES)
                        for r in range(ROUTES):
                            cs = _lane_broadcast(cvs[r], q)
                            for j in range(EMBED // LANES):
                                v = cs * rows[par][r][p, pl.ds(j * LANES,
                                                               LANES)]
                                accs[j] = (v if accs[j] is None
                                           else accs[j] + v)
                        for j in range(EMBED // LANES):
                            pools[par][p, pl.ds(j * LANES, LANES)] = accs[j]

                pltpu.async_copy(pools[par], _out_slice(qt), ssems[par])

        for par in range(2):
            pltpu.make_async_copy(pools[par], _out_slice(nq - 2 + par),
                                  ssems[par]).wait()

    return pool_kernel


def _proj_body(p_ref, w_ref, o_ref):
    o_ref[...] = jnp.dot(p_ref[...], w_ref[...],
                         preferred_element_type=jnp.float32)


def _project(pooled, w_out, num_pos):
    blk = 2048
    return pl.pallas_call(
        _proj_body,
        grid=(num_pos // blk,),
        in_specs=[
            pl.BlockSpec((blk, EMBED), lambda i: (i, 0)),
            pl.BlockSpec((EMBED, HIDDEN), lambda i: (0, 0)),
        ],
        out_specs=pl.BlockSpec((blk, HIDDEN), lambda i: (i, 0)),
        out_shape=jax.ShapeDtypeStruct((num_pos, HIDDEN), jnp.float32),
    )(pooled, w_out)


def kernel(x, W_route, table, W_out):
    B, T, D = x.shape
    num_pos = B * T
    x2 = x.reshape(num_pos, D)
    conf, idx = _routing(x2, W_route, B, T)
    nt = num_pos // 128
    def _rows_view(a):
        return a.reshape(ROUTES, nt, 128).transpose(1, 0, 2).reshape(
            nt * ROUTES, 128)
    pooled = _make_pool_kernel(num_pos)(_rows_view(idx), _rows_view(conf),
                                        table)
    out = _project(pooled, W_out, num_pos)
    return out.reshape(B, T, HIDDEN)


# speedup vs baseline: 1.0508x; 1.0508x over previous
import functools

import jax
import jax.numpy as jnp
from jax import lax
from jax.experimental import pallas as pl
from jax.experimental.pallas import tpu as pltpu
from jax.experimental.pallas import tpu_sc as plsc

HIDDEN = 1024
ROUTES = 8
BITS = 4
NGRAM = 4
ALPHA = 2 ** BITS
EMBED = 128
ROWS = ROUTES * ALPHA ** NGRAM

NUM_CORES = 2
NUM_SUBCORES = 16
NW = NUM_CORES * NUM_SUBCORES
LANES = 16

TILE_POS = 128
QPOS = 32


_RBLK = 2048
_HALO = 8


def _codes_of(logits, sel, gw):
    cb = 1.0 / (1.0 + jnp.exp(-jnp.abs(logits)))
    logcb = jnp.log(cb)
    bits = (logits > 0.0).astype(jnp.float32)
    logconf = jnp.dot(logcb, sel, preferred_element_type=jnp.float32)
    codes = jnp.dot(bits, gw, preferred_element_type=jnp.float32)
    return logconf, codes


def _route_body(x_ref, xh_ref, wr_ref, conf_ref, idx_ref, *, seq):
    T = x_ref.shape[0]
    h = pl.program_id(0) % (seq // _RBLK)
    row = lax.broadcasted_iota(jnp.int32, (ROUTES * BITS, ROUTES), 0)
    col = lax.broadcasted_iota(jnp.int32, (ROUTES * BITS, ROUTES), 1)
    sel = (row // BITS == col).astype(jnp.float32)
    gw = sel * (2.0 ** (row % BITS).astype(jnp.float32))
    logits = jnp.dot(x_ref[...], wr_ref[...],
                     preferred_element_type=jnp.float32)
    logconf, codes = _codes_of(logits, sel, gw)
    hlogits = jnp.dot(xh_ref[...], wr_ref[...],
                      preferred_element_type=jnp.float32)
    _, hcodes = _codes_of(hlogits, sel, gw)
    hcodes = hcodes * jnp.where(h == 0, 0.0, 1.0)
    conf_t = jnp.exp(jnp.transpose(logconf, (1, 0)))
    codes_t = jnp.transpose(codes, (1, 0))
    hcodes_t = jnp.transpose(hcodes, (1, 0))[:, _HALO - NGRAM + 1:]
    ext = jnp.concatenate([hcodes_t, codes_t], axis=1)
    addr = codes_t
    for k in range(1, NGRAM):
        addr = addr + ext[:, NGRAM - 1 - k:NGRAM - 1 - k + T] * float(ALPHA ** k)
    route_off = lax.broadcasted_iota(jnp.int32, (ROUTES, T), 0) * (ALPHA ** NGRAM)
    conf_ref[...] = conf_t
    idx_ref[...] = addr.astype(jnp.int32) + route_off


def _routing(x2, w_route, batch, seq):
    nblk = batch * seq // _RBLK
    per_batch = seq // _RBLK

    def _halo_map(i):
        return (jnp.maximum(i * _RBLK - _HALO, 0) // _HALO, 0)

    return pl.pallas_call(
        functools.partial(_route_body, seq=seq),
        grid=(nblk,),
        in_specs=[
            pl.BlockSpec((_RBLK, HIDDEN), lambda i: (i, 0)),
            pl.BlockSpec((_HALO, HIDDEN), _halo_map),
            pl.BlockSpec((HIDDEN, ROUTES * BITS), lambda i: (0, 0)),
        ],
        out_specs=[
            pl.BlockSpec((ROUTES, _RBLK), lambda i: (0, i)),
            pl.BlockSpec((ROUTES, _RBLK), lambda i: (0, i)),
        ],
        out_shape=[
            jax.ShapeDtypeStruct((ROUTES, batch * seq), jnp.float32),
            jax.ShapeDtypeStruct((ROUTES, batch * seq), jnp.int32),
        ],
    )(x2, x2, w_route)


_GDN = lax.GatherDimensionNumbers(
    offset_dims=(), collapsed_slice_dims=(0,), start_index_map=(0,))


def _lane_broadcast(v, lane):
    idx = jnp.full((LANES, 1), lane, jnp.int32)
    return lax.gather(v, idx, dimension_numbers=_GDN, slice_sizes=(1,),
                      mode=lax.GatherScatterMode.PROMISE_IN_BOUNDS)


def _make_pool_kernel(num_pos):
    pos_w = num_pos // NW
    nrow_w = pos_w * ROUTES // TILE_POS
    nq = pos_w // QPOS
    qper = TILE_POS // QPOS

    mesh = plsc.VectorSubcoreMesh(
        core_axis_name="c", subcore_axis_name="s",
        num_cores=NUM_CORES, num_subcores=NUM_SUBCORES)

    @functools.partial(
        pl.kernel, mesh=mesh,
        out_type=jax.ShapeDtypeStruct((num_pos, EMBED), jnp.float32),
        scratch_types=[
            pltpu.VMEM((nrow_w, TILE_POS), jnp.int32),
            pltpu.VMEM((nrow_w, TILE_POS), jnp.float32),
            [[pltpu.VMEM((QPOS, EMBED), jnp.float32)] * ROUTES] * 2,
            [pltpu.VMEM((QPOS, EMBED), jnp.float32)] * 2,
            [pltpu.SemaphoreType.DMA] * 2,
            [pltpu.SemaphoreType.DMA] * 2,
        ],
    )
    def pool_kernel(idx_hbm, conf_hbm, table_hbm, out_hbm,
                    idx_v, conf_v, rows, pools, gsems, ssems):
        wid = lax.axis_index("s") * NUM_CORES + lax.axis_index("c")
        pltpu.sync_copy(idx_hbm.at[pl.ds(wid * nrow_w, nrow_w)], idx_v)
        pltpu.sync_copy(conf_hbm.at[pl.ds(wid * nrow_w, nrow_w)], conf_v)

        def _gather_quarter(qt, par):
            ct = lax.div(qt, qper)
            off = lax.rem(qt, qper) * QPOS
            for r in range(ROUTES):
                pltpu.async_copy(
                    table_hbm.at[idx_v.at[ct * ROUTES + r, pl.ds(off, QPOS)]],
                    rows[par][r], gsems[par])

        def _drain_quarter(qt, par):
            ct = lax.div(qt, qper)
            off = lax.rem(qt, qper) * QPOS
            for r in range(ROUTES):
                pltpu.make_async_copy(
                    table_hbm.at[idx_v.at[ct * ROUTES + r, pl.ds(off, QPOS)]],
                    rows[par][r], gsems[par]).wait()

        def _out_slice(qt):
            return out_hbm.at[pl.ds(wid * pos_w + qt * QPOS, QPOS)]

        _gather_quarter(0, 0)

        @pl.loop(0, nq, step=2)
        def _qpair(g):
            for par in range(2):
                qt = g + par
                ct = lax.div(qt, qper)
                off = lax.rem(qt, qper) * QPOS

                @pl.when(qt + 1 < nq)
                def _():
                    _gather_quarter(qt + 1, 1 - par)

                _drain_quarter(qt, par)

                @pl.when(qt >= 2)
                def _():
                    pltpu.make_async_copy(pools[par], _out_slice(qt),
                                          ssems[par]).wait()

                @pl.loop(0, QPOS // LANES)
                def _pgroup(pp):
                    cvs = [conf_v[ct * ROUTES + r,
                                  pl.ds(off + pp * LANES, LANES)]
                           for r in range(ROUTES)]

                    @pl.loop(0, LANES)
                    def _pos(q):
                        p = pp * LANES + q
                        accs = [None] * (EMBED // LANES)
                        for r in range(ROUTES):
                            cs = _lane_broadcast(cvs[r], q)
                            for j in range(EMBED // LANES):
                                v = cs * rows[par][r][p, pl.ds(j * LANES,
                                                               LANES)]
                                accs[j] = (v if accs[j] is None
                                           else accs[j] + v)
                        for j in range(EMBED // LANES):
                            pools[par][p, pl.ds(j * LANES, LANES)] = accs[j]

                pltpu.async_copy(pools[par], _out_slice(qt), ssems[par])

        for par in range(2):
            pltpu.make_async_copy(pools[par], _out_slice(nq - 2 + par),
                                  ssems[par]).wait()

    return pool_kernel


def _proj_body(p_ref, w_ref, o_ref):
    o_ref[...] = jnp.dot(p_ref[...], w_ref[...],
                         preferred_element_type=jnp.float32)


def _project(pooled, w_out, num_pos):
    blk = 2048
    return pl.pallas_call(
        _proj_body,
        grid=(num_pos // blk,),
        in_specs=[
            pl.BlockSpec((blk, EMBED), lambda i: (i, 0)),
            pl.BlockSpec((EMBED, HIDDEN), lambda i: (0, 0)),
        ],
        out_specs=pl.BlockSpec((blk, HIDDEN), lambda i: (i, 0)),
        out_shape=jax.ShapeDtypeStruct((num_pos, HIDDEN), jnp.float32),
    )(pooled, w_out)


def kernel(x, W_route, table, W_out):
    B, T, D = x.shape
    num_pos = B * T
    x2 = x.reshape(num_pos, D)
    conf, idx = _routing(x2, W_route, B, T)
    nt = num_pos // 128
    def _rows_view(a):
        return a.reshape(ROUTES, nt, 128).transpose(1, 0, 2).reshape(
            nt * ROUTES, 128)
    pooled = _make_pool_kernel(num_pos)(_rows_view(idx), _rows_view(conf),
                                        table)
    out = _project(pooled, W_out, num_pos)
    return out.reshape(B, T, HIDDEN)


# speedup vs baseline: 1.0580x; 1.0068x over previous
import functools

import jax
import jax.numpy as jnp
from jax import lax
from jax.experimental import pallas as pl
from jax.experimental.pallas import tpu as pltpu
from jax.experimental.pallas import tpu_sc as plsc

HIDDEN = 1024
ROUTES = 8
BITS = 4
NGRAM = 4
ALPHA = 2 ** BITS
EMBED = 128
ROWS = ROUTES * ALPHA ** NGRAM

NUM_CORES = 2
NUM_SUBCORES = 16
NW = NUM_CORES * NUM_SUBCORES
LANES = 16

TILE_POS = 128
QPOS = 32


def _route_body(x_ref, wr_ref, conf_ref, idx_ref):
    T = x_ref.shape[0]
    logits = jnp.dot(x_ref[...], wr_ref[...],
                     preferred_element_type=jnp.float32)
    cb = 1.0 / (1.0 + jnp.exp(-jnp.abs(logits)))
    logcb = jnp.log(cb)
    bits = (logits > 0.0).astype(jnp.float32)
    row = lax.broadcasted_iota(jnp.int32, (ROUTES * BITS, ROUTES), 0)
    col = lax.broadcasted_iota(jnp.int32, (ROUTES * BITS, ROUTES), 1)
    sel = (row // BITS == col).astype(jnp.float32)
    gw = sel * (2.0 ** (row % BITS).astype(jnp.float32))
    logconf = jnp.dot(logcb, sel, preferred_element_type=jnp.float32)
    codes = jnp.dot(bits, gw, preferred_element_type=jnp.float32)
    conf_t = jnp.exp(jnp.transpose(logconf, (1, 0)))
    codes_t = jnp.transpose(codes, (1, 0))
    addr = codes_t
    zcol = jnp.zeros((ROUTES, 1), jnp.float32)
    shifted = codes_t
    for k in range(1, NGRAM):
        shifted = jnp.concatenate([zcol, shifted[:, :T - 1]], axis=1)
        addr = addr + shifted * float(ALPHA ** k)
    route_off = lax.broadcasted_iota(jnp.int32, (ROUTES, T), 0) * (ALPHA ** NGRAM)
    conf_ref[...] = conf_t
    idx_ref[...] = addr.astype(jnp.int32) + route_off


def _routing(x2, w_route, batch, seq):
    return pl.pallas_call(
        _route_body,
        grid=(batch,),
        in_specs=[
            pl.BlockSpec((seq, HIDDEN), lambda b: (b, 0)),
            pl.BlockSpec((HIDDEN, ROUTES * BITS), lambda b: (0, 0)),
        ],
        out_specs=[
            pl.BlockSpec((ROUTES, seq), lambda b: (0, b)),
            pl.BlockSpec((ROUTES, seq), lambda b: (0, b)),
        ],
        out_shape=[
            jax.ShapeDtypeStruct((ROUTES, batch * seq), jnp.float32),
            jax.ShapeDtypeStruct((ROUTES, batch * seq), jnp.int32),
        ],
    )(x2, w_route)


_GDN = lax.GatherDimensionNumbers(
    offset_dims=(), collapsed_slice_dims=(0,), start_index_map=(0,))


def _lane_broadcast(v, lane):
    idx = jnp.full((LANES, 1), lane, jnp.int32)
    return lax.gather(v, idx, dimension_numbers=_GDN, slice_sizes=(1,),
                      mode=lax.GatherScatterMode.PROMISE_IN_BOUNDS)


def _make_pool_kernel(num_pos):
    pos_w = num_pos // NW
    nrow_w = pos_w * ROUTES // TILE_POS
    nq = pos_w // QPOS
    qper = TILE_POS // QPOS

    mesh = plsc.VectorSubcoreMesh(
        core_axis_name="c", subcore_axis_name="s",
        num_cores=NUM_CORES, num_subcores=NUM_SUBCORES)

    @functools.partial(
        pl.kernel, mesh=mesh,
        out_type=jax.ShapeDtypeStruct((num_pos, EMBED), jnp.float32),
        scratch_types=[
            pltpu.VMEM((nrow_w, TILE_POS), jnp.int32),
            pltpu.VMEM((nrow_w, TILE_POS), jnp.float32),
            [[pltpu.VMEM((QPOS, EMBED), jnp.float32)] * ROUTES] * 2,
            [pltpu.VMEM((QPOS, EMBED), jnp.float32)] * 2,
            [pltpu.SemaphoreType.DMA] * 2,
            [pltpu.SemaphoreType.DMA] * 2,
        ],
    )
    def pool_kernel(idx_hbm, conf_hbm, table_hbm, out_hbm,
                    idx_v, conf_v, rows, pools, gsems, ssems):
        wid = lax.axis_index("s") * NUM_CORES + lax.axis_index("c")
        pltpu.sync_copy(idx_hbm.at[pl.ds(wid * nrow_w, nrow_w)], idx_v)
        pltpu.sync_copy(conf_hbm.at[pl.ds(wid * nrow_w, nrow_w)], conf_v)

        def _gather_quarter(qt, par):
            ct = lax.div(qt, qper)
            off = lax.rem(qt, qper) * QPOS
            for r in range(ROUTES):
                pltpu.async_copy(
                    table_hbm.at[idx_v.at[ct * ROUTES + r, pl.ds(off, QPOS)]],
                    rows[par][r], gsems[par])

        def _drain_quarter(qt, par):
            ct = lax.div(qt, qper)
            off = lax.rem(qt, qper) * QPOS
            for r in range(ROUTES):
                pltpu.make_async_copy(
                    table_hbm.at[idx_v.at[ct * ROUTES + r, pl.ds(off, QPOS)]],
                    rows[par][r], gsems[par]).wait()

        def _out_slice(qt):
            return out_hbm.at[pl.ds(wid * pos_w + qt * QPOS, QPOS)]

        _gather_quarter(0, 0)

        @pl.loop(0, nq, step=2)
        def _qpair(g):
            for par in range(2):
                qt = g + par
                ct = lax.div(qt, qper)
                off = lax.rem(qt, qper) * QPOS

                @pl.when(qt + 1 < nq)
                def _():
                    _gather_quarter(qt + 1, 1 - par)

                _drain_quarter(qt, par)

                @pl.when(qt >= 2)
                def _():
                    pltpu.make_async_copy(pools[par], _out_slice(qt),
                                          ssems[par]).wait()

                @pl.loop(0, QPOS // LANES)
                def _pgroup(pp):
                    cvs = [conf_v[ct * ROUTES + r,
                                  pl.ds(off + pp * LANES, LANES)]
                           for r in range(ROUTES)]

                    @pl.loop(0, LANES)
                    def _pos(q):
                        p = pp * LANES + q
                        accs = [None] * (EMBED // LANES)
                        for r in range(ROUTES):
                            cs = _lane_broadcast(cvs[r], q)
                            for j in range(EMBED // LANES):
                                v = cs * rows[par][r][p, pl.ds(j * LANES,
                                                               LANES)]
                                accs[j] = (v if accs[j] is None
                                           else accs[j] + v)
                        for j in range(EMBED // LANES):
                            pools[par][p, pl.ds(j * LANES, LANES)] = accs[j]

                pltpu.async_copy(pools[par], _out_slice(qt), ssems[par])

        for par in range(2):
            pltpu.make_async_copy(pools[par], _out_slice(nq - 2 + par),
                                  ssems[par]).wait()

    return pool_kernel


def _proj_body(p_ref, w_ref, o_ref):
    o_ref[...] = jnp.dot(p_ref[...], w_ref[...],
                         preferred_element_type=jnp.float32)


def _project(pooled, w_out, num_pos):
    blk = 2048
    return pl.pallas_call(
        _proj_body,
        grid=(num_pos // blk,),
        in_specs=[
            pl.BlockSpec((blk, EMBED), lambda i: (i, 0)),
            pl.BlockSpec((EMBED, HIDDEN), lambda i: (0, 0)),
        ],
        out_specs=pl.BlockSpec((blk, HIDDEN), lambda i: (i, 0)),
        out_shape=jax.ShapeDtypeStruct((num_pos, HIDDEN), jnp.float32),
    )(pooled, w_out)


def kernel(x, W_route, table, W_out):
    B, T, D = x.shape
    num_pos = B * T
    x2 = x.reshape(num_pos, D)
    conf, idx = _routing(x2, W_route, B, T)
    nt = num_pos // 128
    def _rows_view(a):
        return a.reshape(ROUTES, nt, 128).transpose(1, 0, 2).reshape(
            nt * ROUTES, 128)
    pooled = _make_pool_kernel(num_pos)(_rows_view(idx), _rows_view(conf),
                                        table)
    out = _project(pooled, W_out, num_pos)
    return out.reshape(B, T, HIDDEN)


# speedup vs baseline: 1.0933x; 1.0334x over previous
import functools

import jax
import jax.numpy as jnp
from jax import lax
from jax.experimental import pallas as pl
from jax.experimental.pallas import tpu as pltpu
from jax.experimental.pallas import tpu_sc as plsc

HIDDEN = 1024
ROUTES = 8
BITS = 4
NGRAM = 4
ALPHA = 2 ** BITS
EMBED = 128
ROWS = ROUTES * ALPHA ** NGRAM

NUM_CORES = 2
NUM_SUBCORES = 16
NW = NUM_CORES * NUM_SUBCORES
LANES = 16

TILE_POS = 128
QPOS = 16
NBUF = 4


def _route_body(x_ref, wr_ref, conf_ref, idx_ref):
    T = x_ref.shape[0]
    logits = jnp.dot(x_ref[...], wr_ref[...],
                     preferred_element_type=jnp.float32)
    cb = 1.0 / (1.0 + jnp.exp(-jnp.abs(logits)))
    logcb = jnp.log(cb)
    bits = (logits > 0.0).astype(jnp.float32)
    row = lax.broadcasted_iota(jnp.int32, (ROUTES * BITS, ROUTES), 0)
    col = lax.broadcasted_iota(jnp.int32, (ROUTES * BITS, ROUTES), 1)
    sel = (row // BITS == col).astype(jnp.float32)
    gw = sel * (2.0 ** (row % BITS).astype(jnp.float32))
    logconf = jnp.dot(logcb, sel, preferred_element_type=jnp.float32)
    codes = jnp.dot(bits, gw, preferred_element_type=jnp.float32)
    conf_t = jnp.exp(jnp.transpose(logconf, (1, 0)))
    codes_t = jnp.transpose(codes, (1, 0))
    addr = codes_t
    zcol = jnp.zeros((ROUTES, 1), jnp.float32)
    shifted = codes_t
    for k in range(1, NGRAM):
        shifted = jnp.concatenate([zcol, shifted[:, :T - 1]], axis=1)
        addr = addr + shifted * float(ALPHA ** k)
    route_off = lax.broadcasted_iota(jnp.int32, (ROUTES, T), 0) * (ALPHA ** NGRAM)
    conf_ref[...] = conf_t
    idx_ref[...] = addr.astype(jnp.int32) + route_off


def _routing(x2, w_route, batch, seq):
    return pl.pallas_call(
        _route_body,
        grid=(batch,),
        in_specs=[
            pl.BlockSpec((seq, HIDDEN), lambda b: (b, 0)),
            pl.BlockSpec((HIDDEN, ROUTES * BITS), lambda b: (0, 0)),
        ],
        out_specs=[
            pl.BlockSpec((ROUTES, seq), lambda b: (0, b)),
            pl.BlockSpec((ROUTES, seq), lambda b: (0, b)),
        ],
        out_shape=[
            jax.ShapeDtypeStruct((ROUTES, batch * seq), jnp.float32),
            jax.ShapeDtypeStruct((ROUTES, batch * seq), jnp.int32),
        ],
    )(x2, w_route)


_GDN = lax.GatherDimensionNumbers(
    offset_dims=(), collapsed_slice_dims=(0,), start_index_map=(0,))


def _lane_broadcast(v, lane):
    idx = jnp.full((LANES, 1), lane, jnp.int32)
    return lax.gather(v, idx, dimension_numbers=_GDN, slice_sizes=(1,),
                      mode=lax.GatherScatterMode.PROMISE_IN_BOUNDS)


def _make_pool_kernel(num_pos):
    pos_w = num_pos // NW
    nrow_w = pos_w * ROUTES // TILE_POS
    nq = pos_w // QPOS
    qper = TILE_POS // QPOS

    mesh = plsc.VectorSubcoreMesh(
        core_axis_name="c", subcore_axis_name="s",
        num_cores=NUM_CORES, num_subcores=NUM_SUBCORES)

    @functools.partial(
        pl.kernel, mesh=mesh,
        out_type=jax.ShapeDtypeStruct((num_pos, EMBED), jnp.float32),
        scratch_types=[
            pltpu.VMEM((nrow_w, TILE_POS), jnp.int32),
            pltpu.VMEM((nrow_w, TILE_POS), jnp.float32),
            [[pltpu.VMEM((QPOS, EMBED), jnp.float32)] * ROUTES] * NBUF,
            [pltpu.VMEM((QPOS, EMBED), jnp.float32)] * NBUF,
            [pltpu.SemaphoreType.DMA] * NBUF,
            [pltpu.SemaphoreType.DMA] * NBUF,
        ],
    )
    def pool_kernel(idx_hbm, conf_hbm, table_hbm, out_hbm,
                    idx_v, conf_v, rows, pools, gsems, ssems):
        wid = lax.axis_index("s") * NUM_CORES + lax.axis_index("c")
        pltpu.sync_copy(idx_hbm.at[pl.ds(wid * nrow_w, nrow_w)], idx_v)
        pltpu.sync_copy(conf_hbm.at[pl.ds(wid * nrow_w, nrow_w)], conf_v)

        def _gather_quarter(qt, par):
            ct = lax.div(qt, qper)
            off = lax.rem(qt, qper) * QPOS
            for r in range(ROUTES):
                pltpu.async_copy(
                    table_hbm.at[idx_v.at[ct * ROUTES + r, pl.ds(off, QPOS)]],
                    rows[par][r], gsems[par])

        def _drain_quarter(qt, par):
            ct = lax.div(qt, qper)
            off = lax.rem(qt, qper) * QPOS
            for r in range(ROUTES):
                pltpu.make_async_copy(
                    table_hbm.at[idx_v.at[ct * ROUTES + r, pl.ds(off, QPOS)]],
                    rows[par][r], gsems[par]).wait()

        def _out_slice(qt):
            return out_hbm.at[pl.ds(wid * pos_w + qt * QPOS, QPOS)]

        for b in range(NBUF - 1):
            _gather_quarter(b, b)

        @pl.loop(0, nq, step=NBUF)
        def _qgroup(g):
            for par in range(NBUF):
                qt = g + par
                ct = lax.div(qt, qper)
                off = lax.rem(qt, qper) * QPOS

                @pl.when(qt + NBUF - 1 < nq)
                def _():
                    _gather_quarter(qt + NBUF - 1, (par + NBUF - 1) % NBUF)

                _drain_quarter(qt, par)

                @pl.when(qt >= NBUF)
                def _():
                    pltpu.make_async_copy(pools[par], _out_slice(qt),
                                          ssems[par]).wait()

                @pl.loop(0, QPOS // LANES)
                def _pgroup(pp):
                    cvs = [conf_v[ct * ROUTES + r,
                                  pl.ds(off + pp * LANES, LANES)]
                           for r in range(ROUTES)]

                    @pl.loop(0, LANES)
                    def _pos(q):
                        p = pp * LANES + q
                        accs = [None] * (EMBED // LANES)
                        for r in range(ROUTES):
                            cs = _lane_broadcast(cvs[r], q)
                            for j in range(EMBED // LANES):
                                v = cs * rows[par][r][p, pl.ds(j * LANES,
                                                               LANES)]
                                accs[j] = (v if accs[j] is None
                                           else accs[j] + v)
                        for j in range(EMBED // LANES):
                            pools[par][p, pl.ds(j * LANES, LANES)] = accs[j]

                pltpu.async_copy(pools[par], _out_slice(qt), ssems[par])

        for par in range(NBUF):
            pltpu.make_async_copy(pools[par], _out_slice(nq - NBUF + par),
                                  ssems[par]).wait()

    return pool_kernel


def _proj_body(p_ref, w_ref, o_ref):
    o_ref[...] = jnp.dot(p_ref[...], w_ref[...],
                         preferred_element_type=jnp.float32)


def _project(pooled, w_out, num_pos):
    blk = 2048
    return pl.pallas_call(
        _proj_body,
        grid=(num_pos // blk,),
        in_specs=[
            pl.BlockSpec((blk, EMBED), lambda i: (i, 0)),
            pl.BlockSpec((EMBED, HIDDEN), lambda i: (0, 0)),
        ],
        out_specs=pl.BlockSpec((blk, HIDDEN), lambda i: (i, 0)),
        out_shape=jax.ShapeDtypeStruct((num_pos, HIDDEN), jnp.float32),
    )(pooled, w_out)


def kernel(x, W_route, table, W_out):
    B, T, D = x.shape
    num_pos = B * T
    x2 = x.reshape(num_pos, D)
    conf, idx = _routing(x2, W_route, B, T)
    nt = num_pos // 128
    def _rows_view(a):
        return a.reshape(ROUTES, nt, 128).transpose(1, 0, 2).reshape(
            nt * ROUTES, 128)
    pooled = _make_pool_kernel(num_pos)(_rows_view(idx), _rows_view(conf),
                                        table)
    out = _project(pooled, W_out, num_pos)
    return out.reshape(B, T, HIDDEN)
